# Initial kernel scaffold; baseline (speedup 1.0000x reference)
import jax
import jax.numpy as jnp
from jax.experimental import pallas as pl


def kernel(inputs, hx, edge_w, W_fc, b_fc, W_g, b_g, edge_src, edge_dst):
    raise NotImplementedError("write your pallas kernel here")


# speedup vs baseline: 1.3590x; 1.3590x over previous
import functools

import jax
import jax.numpy as jnp
from jax import lax
from jax.experimental import pallas as pl
from jax.experimental.pallas import tpu as pltpu
from jax.experimental.pallas import tpu_sc as plsc

N = 10000
E = 160000
IN_DIM = 2
UNITS = 32
B = 16
NUM_MATRICES = 5
IS = IN_DIM + UNITS
ISP = 40
NQ = 8
BQ = B // NQ
FQ = BQ * ISP
NQC = NQ // 2
NPAD = 10240
NTILES = 16
NROW = NPAD // NTILES
EB = 128
NB_E = 80
ET = NB_E * EB
EPAD = NTILES * ET
NSUB = NROW // EB
NVR = FQ // 16

NBLK = 400
NBLOCKS = N // NBLK


def _gates_body(inp_ref, hx_ref, wfc_ref, bfc_ref, x0_ref, u_ref):
    wfc = wfc_ref[...]
    bfc = bfc_ref[...]
    zpad = jnp.zeros((NBLK, ISP - IS), jnp.float32)
    for b in range(B):
        inp_b = inp_ref[b]
        hx_b = hx_ref[b]
        cat = jnp.concatenate([inp_b, hx_b], axis=1)
        v = jax.nn.sigmoid(
            jnp.dot(cat, wfc, preferred_element_type=jnp.float32) + bfc)
        r = v[:, :UNITS]
        u_ref[b] = v[:, UNITS:]
        ias = jnp.concatenate([inp_b, r * hx_b, zpad], axis=1)
        q, bq = divmod(b, BQ)
        x0_ref[q, :, bq * ISP:(bq + 1) * ISP] = ias


def _gates(inputs_r, hx_r, w_fc, b_fc2):
    return pl.pallas_call(
        _gates_body,
        grid=(NBLOCKS,),
        in_specs=[
            pl.BlockSpec((B, NBLK, IN_DIM), lambda j: (0, j, 0)),
            pl.BlockSpec((B, NBLK, UNITS), lambda j: (0, j, 0)),
            pl.BlockSpec((IS, 2 * UNITS), lambda j: (0, 0)),
            pl.BlockSpec((1, 2 * UNITS), lambda j: (0, 0)),
        ],
        out_specs=[
            pl.BlockSpec((NQ, NBLK, FQ), lambda j: (0, j, 0)),
            pl.BlockSpec((B, NBLK, UNITS), lambda j: (0, j, 0)),
        ],
        out_shape=[
            jax.ShapeDtypeStruct((NQ, NPAD, FQ), jnp.float32),
            jax.ShapeDtypeStruct((B, N, UNITS), jnp.float32),
        ],
    )(inputs_r, hx_r, w_fc, b_fc2)


def _sc_degvals(src_t, dst_t, w_t):
    mesh = plsc.VectorSubcoreMesh(core_axis_name="c", subcore_axis_name="s")
    out_type = [jax.ShapeDtypeStruct((NTILES, NB_E, EB), jnp.float32)
                for _ in range(2)]
    scratch = [
        pltpu.VMEM((NB_E, EB), jnp.int32),
        pltpu.VMEM((NB_E, EB), jnp.int32),
        pltpu.VMEM((NB_E, EB), jnp.float32),
        pltpu.VMEM((NB_E, EB), jnp.float32),
        pltpu.VMEM((NB_E, EB), jnp.float32),
        pltpu.VMEM((EB, 16), jnp.float32),
        pltpu.VMEM((NROW, 16), jnp.float32),
        pltpu.VMEM((NPAD,), jnp.float32),
        pltpu.VMEM((NPAD,), jnp.float32),
        pltpu.VMEM_SHARED((NPAD, 16), jnp.float32),
        pltpu.VMEM_SHARED((NPAD, 16), jnp.float32),
        pltpu.VMEM_SHARED((NPAD,), jnp.float32),
        pltpu.VMEM_SHARED((NPAD,), jnp.float32),
    ]

    @functools.partial(
        pl.kernel, out_type=out_type, mesh=mesh, scratch_types=scratch,
        compiler_params=pltpu.CompilerParams(
            needs_layout_passes=False, use_tc_tiling_on_sc=False))
    def ka(src_h, dst_h, w_h, val1_h, val2_h,
           src_v, dst_v, val1_v, val2_v, w_v, zrow_v, dcol_v,
           dinvo_v, dinvi_v, deg2o, deg2i, dinv_o_sh, dinv_i_sh):
        c = lax.axis_index("c")
        s = lax.axis_index("s")
        iota16 = lax.iota(jnp.int32, 16)
        zc16 = jnp.zeros((16,), jnp.int32)
        z16 = jnp.zeros((16,), jnp.float32)
        pltpu.sync_copy(src_h.at[s], src_v)
        pltpu.sync_copy(dst_h.at[s], dst_v)

        if True:
            pltpu.sync_copy(w_h.at[s], w_v)

            def zrow_body(e, _):
                zrow_v[e, :] = z16
                return _
            lax.fori_loop(0, EB, zrow_body, None)
            row0 = s * NROW
            for t in range(NSUB):
                pltpu.sync_copy(zrow_v, deg2o.at[pl.ds(row0 + t * EB, EB)])
                pltpu.sync_copy(zrow_v, deg2i.at[pl.ds(row0 + t * EB, EB)])
            plsc.subcore_barrier()

            def degj(j, _):
                def dege(e16, _):
                    wv = w_v[j, pl.ds(e16 * 16, 16)]
                    ridx = e16 * 16 + iota16
                    plsc.store_scatter(zrow_v, [ridx, zc16], wv)
                    return _
                lax.fori_loop(0, EB // 16, dege, _)
                pltpu.sync_copy(zrow_v, deg2o.at[src_v.at[j]], add=True)
                pltpu.sync_copy(zrow_v, deg2i.at[dst_v.at[j]], add=True)
                return _
            lax.fori_loop(0, NB_E, degj, None)
            plsc.subcore_barrier()

            def invert(deg2, dinv_sh, dinv_v):
                pltpu.sync_copy(deg2.at[pl.ds(row0, NROW)], dcol_v)

                def cj(jv, _):
                    d = plsc.load_gather(dcol_v, [jv * 16 + iota16, zc16])
                    dv = jnp.where(d > 0.0, 1.0 / d, 0.0)
                    dinv_v[pl.ds(row0 + jv * 16, 16)] = dv
                    return _
                lax.fori_loop(0, NROW // 16, cj, None)
                pltpu.sync_copy(dinv_v.at[pl.ds(row0, NROW)],
                                dinv_sh.at[pl.ds(row0, NROW)])

            invert(deg2o, dinv_o_sh, dinvo_v)
            invert(deg2i, dinv_i_sh, dinvi_v)
            plsc.subcore_barrier()
            pltpu.sync_copy(dinv_o_sh, dinvo_v)
            pltpu.sync_copy(dinv_i_sh, dinvi_v)

            def vj(j, _):
                def ve(ev, _):
                    sl = pl.ds(ev * 16, 16)
                    si = src_v[j, sl]
                    di = dst_v[j, sl]
                    wv = w_v[j, sl]
                    val1_v[j, sl] = wv * plsc.load_gather(dinvo_v, [si])
                    val2_v[j, sl] = wv * plsc.load_gather(dinvi_v, [di])
                    return _
                return lax.fori_loop(0, EB // 16, ve, _)
            lax.fori_loop(0, NB_E, vj, None)

            @pl.when(c == 0)
            def _():
                pltpu.sync_copy(val1_v, val1_h.at[s])
                pltpu.sync_copy(val2_v, val2_h.at[s])

    return ka(src_t, dst_t, w_t)


def _sc_spmm(x0, src_t, dst_t, val1_t, val2_t):
    mesh = plsc.VectorSubcoreMesh(core_axis_name="c", subcore_axis_name="s")
    out_type = [jax.ShapeDtypeStruct((NQ, NPAD, FQ), jnp.float32)
                for _ in range(4)]
    scratch = [
        pltpu.VMEM((NB_E, EB), jnp.int32),
        pltpu.VMEM((NB_E, EB), jnp.int32),
        pltpu.VMEM((NB_E, EB), jnp.float32),
        pltpu.VMEM((NB_E, EB), jnp.float32),
        pltpu.VMEM((EB, FQ), jnp.float32),
        pltpu.VMEM((EB, FQ), jnp.float32),
        pltpu.VMEM_SHARED((NPAD, FQ), jnp.float32),
        pltpu.SemaphoreType.DMA,
        pltpu.SemaphoreType.DMA,
    ]

    @functools.partial(
        pl.kernel, out_type=out_type, mesh=mesh, scratch_types=scratch,
        compiler_params=pltpu.CompilerParams(
            needs_layout_passes=False, use_tc_tiling_on_sc=False))
    def kb(x0_h, src_h, dst_h, val1_h, val2_h, m1_h, m2_h, m3_h, m4_h,
           src_v, dst_v, val1_v, val2_v, buf_a, buf_b, acc, sem_a, sem_b):
        c = lax.axis_index("c")
        s = lax.axis_index("s")
        iota16 = lax.iota(jnp.int32, 16)
        z16 = jnp.zeros((16,), jnp.float32)
        pltpu.sync_copy(src_h.at[s], src_v)
        pltpu.sync_copy(dst_h.at[s], dst_v)
        pltpu.sync_copy(val1_h.at[s], val1_v)
        pltpu.sync_copy(val2_h.at[s], val2_v)

        if True:
            def zero_rows(e, _):
                for rsub in range(NVR):
                    buf_a[e, pl.ds(rsub * 16, 16)] = z16
                return _

            def scale_scatter(buf, val_v, double, gi):
                def sbody(e16, _):
                    v = val_v[gi, pl.ds(e16 * 16, 16)]
                    if double:
                        v = v + v
                    ridx = e16 * 16 + iota16

                    def cbody(cg, _):
                        for cc in range(16):
                            cidx = lax.broadcast(cg * 16 + cc, (16,))
                            g = plsc.load_gather(buf, [ridx, cidx])
                            plsc.store_scatter(buf, [ridx, cidx], g * v)
                        return _
                    return lax.fori_loop(0, NVR, cbody, _)
                lax.fori_loop(0, EB // 16, sbody, None)

            def do_pass(in_hq, gidx_v, sidx_v, val_v, double, out_hq, sub_hq):
                lax.fori_loop(0, EB, zero_rows, None)
                row0 = s * NROW
                for t in range(NSUB):
                    pltpu.sync_copy(buf_a, acc.at[pl.ds(row0 + t * EB, EB)])
                plsc.subcore_barrier()

                def gather(g, buf, sem):
                    return pltpu.make_async_copy(
                        in_hq.at[gidx_v.at[g]], buf, sem)

                gather(0, buf_a, sem_a).start()

                def pair(g2, _):
                    g = g2 * 2
                    gather(g, buf_a, sem_a).wait()
                    gather(g + 1, buf_b, sem_b).start()
                    scale_scatter(buf_a, val_v, double, g)
                    pltpu.sync_copy(buf_a, acc.at[sidx_v.at[g]], add=True)
                    gather(g + 1, buf_b, sem_b).wait()

                    @pl.when(g + 2 < NB_E)
                    def _():
                        gather(g + 2, buf_a, sem_a).start()
                    scale_scatter(buf_b, val_v, double, g + 1)
                    pltpu.sync_copy(buf_b, acc.at[sidx_v.at[g + 1]], add=True)
                    return _
                lax.fori_loop(0, NB_E // 2, pair, None)
                plsc.subcore_barrier()

                for t in range(NSUB):
                    r0 = row0 + t * EB
                    pltpu.sync_copy(acc.at[pl.ds(r0, EB)], buf_a)
                    if sub_hq is not None:
                        pltpu.sync_copy(sub_hq.at[pl.ds(r0, EB)], buf_b)

                        def sub_body(e, _):
                            for rsub in range(NVR):
                                sl = pl.ds(rsub * 16, 16)
                                buf_a[e, sl] = buf_a[e, sl] - buf_b[e, sl]
                            return _
                        lax.fori_loop(0, EB, sub_body, None)
                    pltpu.sync_copy(buf_a, out_hq.at[pl.ds(r0, EB)])
                plsc.subcore_barrier()

            def chunk(j, _):
                q = c * NQC + j
                x0q = x0_h.at[q]
                m1q = m1_h.at[q]
                m2q = m2_h.at[q]
                m3q = m3_h.at[q]
                m4q = m4_h.at[q]
                do_pass(x0q, src_v, dst_v, val1_v, False, m1q, None)
                do_pass(m1q, src_v, dst_v, val1_v, True, m2q, x0q)
                do_pass(m1q, dst_v, src_v, val2_v, False, m3q, None)
                do_pass(m3q, dst_v, src_v, val2_v, True, m4q, m1q)
                return _
            lax.fori_loop(0, NQC, chunk, None)

    return kb(x0, src_t, dst_t, val1_t, val2_t)


def _out_body(x0_ref, m1_ref, m2_ref, m3_ref, m4_ref, u_ref, hx_ref,
              wcat_ref, bg_ref, out_ref):
    wcat = wcat_ref[...]
    bg = bg_ref[...]
    for b in range(B):
        q, bq = divmod(b, BQ)
        c0 = bq * ISP
        mcat = jnp.concatenate(
            [x0_ref[q][:, c0:c0 + IS], m1_ref[q][:, c0:c0 + IS],
             m2_ref[q][:, c0:c0 + IS], m3_ref[q][:, c0:c0 + IS],
             m4_ref[q][:, c0:c0 + IS]], axis=1)
        cc = jnp.tanh(
            jnp.dot(mcat, wcat, preferred_element_type=jnp.float32) + bg)
        u_b = u_ref[b]
        hx_b = hx_ref[b]
        out_ref[b] = u_b * hx_b + (1.0 - u_b) * cc


def _output(x0, m1, m2, m3, m4, u3, hx_r, w_cat, b_g2):
    mspec = pl.BlockSpec((NQ, NBLK, FQ), lambda j: (0, j, 0))
    bspec = pl.BlockSpec((B, NBLK, UNITS), lambda j: (0, j, 0))
    return pl.pallas_call(
        _out_body,
        grid=(NBLOCKS,),
        in_specs=[mspec, mspec, mspec, mspec, mspec, bspec, bspec,
                  pl.BlockSpec((NUM_MATRICES * IS, UNITS), lambda j: (0, 0)),
                  pl.BlockSpec((1, UNITS), lambda j: (0, 0))],
        out_specs=bspec,
        out_shape=jax.ShapeDtypeStruct((B, N, UNITS), jnp.float32),
    )(x0, m1, m2, m3, m4, u3, hx_r, w_cat, b_g2)


def kernel(inputs, hx, edge_w, W_fc, b_fc, W_g, b_g, edge_src, edge_dst):
    inputs_r = inputs.reshape(B, N, IN_DIM)
    hx_r = hx.reshape(B, N, UNITS)
    b_fc2 = b_fc.reshape(1, 2 * UNITS)
    b_g2 = b_g.reshape(1, UNITS)
    w_cat = W_g.reshape(IS, NUM_MATRICES, UNITS).transpose(1, 0, 2)
    w_cat = w_cat.reshape(NUM_MATRICES * IS, UNITS)

    npad_e = EPAD - E
    pad_idx = (jnp.arange(npad_e, dtype=jnp.int32) * 41) % N
    src_t = jnp.concatenate([edge_src.astype(jnp.int32), pad_idx])
    src_t = src_t.reshape(NTILES, NB_E, EB)
    dst_t = jnp.concatenate([edge_dst.astype(jnp.int32), pad_idx])
    dst_t = dst_t.reshape(NTILES, NB_E, EB)
    w_t = jnp.concatenate(
        [edge_w, jnp.zeros((npad_e,), jnp.float32)]).reshape(NTILES, NB_E, EB)

    x0, u3 = _gates(inputs_r, hx_r, W_fc, b_fc2)
    val1_t, val2_t = _sc_degvals(src_t, dst_t, w_t)
    m1, m2, m3, m4 = _sc_spmm(x0, src_t, dst_t, val1_t, val2_t)
    out3 = _output(x0, m1, m2, m3, m4, u3, hx_r, w_cat, b_g2)
    return out3.reshape(B, N * UNITS)


# speedup vs baseline: 3.5466x; 2.6098x over previous
import functools

import jax
import jax.numpy as jnp
from jax import lax
from jax.experimental import pallas as pl
from jax.experimental.pallas import tpu as pltpu
from jax.experimental.pallas import tpu_sc as plsc

N = 10000
E = 160000
IN_DIM = 2
UNITS = 32
B = 16
NUM_MATRICES = 5
IS = IN_DIM + UNITS
ISP = 40
NQ = 8
BQ = B // NQ
FQ = BQ * ISP
NQC = NQ // 2
NPAD = 10240
NTILES = 16
NROW = NPAD // NTILES
EB = 128
NB_E = 80
ET = NB_E * EB
EPAD = NTILES * ET
NSUB = NROW // EB
NVR = FQ // 16

NBLK = 400
NBLOCKS = N // NBLK


def _gates_body(inp_ref, hx_ref, wfc_ref, bfc_ref, x0_ref, u_ref):
    wfc = wfc_ref[...]
    bfc = bfc_ref[...]
    zpad = jnp.zeros((NBLK, ISP - IS), jnp.float32)
    for b in range(B):
        inp_b = inp_ref[b]
        hx_b = hx_ref[b]
        cat = jnp.concatenate([inp_b, hx_b], axis=1)
        v = jax.nn.sigmoid(
            jnp.dot(cat, wfc, preferred_element_type=jnp.float32) + bfc)
        r = v[:, :UNITS]
        u_ref[b] = v[:, UNITS:]
        ias = jnp.concatenate([inp_b, r * hx_b, zpad], axis=1)
        q, bq = divmod(b, BQ)
        x0_ref[q, :, bq * ISP:(bq + 1) * ISP] = ias


def _gates(inputs_r, hx_r, w_fc, b_fc2):
    return pl.pallas_call(
        _gates_body,
        grid=(NBLOCKS,),
        in_specs=[
            pl.BlockSpec((B, NBLK, IN_DIM), lambda j: (0, j, 0)),
            pl.BlockSpec((B, NBLK, UNITS), lambda j: (0, j, 0)),
            pl.BlockSpec((IS, 2 * UNITS), lambda j: (0, 0)),
            pl.BlockSpec((1, 2 * UNITS), lambda j: (0, 0)),
        ],
        out_specs=[
            pl.BlockSpec((NQ, NBLK, FQ), lambda j: (0, j, 0)),
            pl.BlockSpec((B, NBLK, UNITS), lambda j: (0, j, 0)),
        ],
        out_shape=[
            jax.ShapeDtypeStruct((NQ, NPAD, FQ), jnp.float32),
            jax.ShapeDtypeStruct((B, N, UNITS), jnp.float32),
        ],
    )(inputs_r, hx_r, w_fc, b_fc2)


def _sc_degvals(src_t, dst_t, w_t):
    mesh = plsc.VectorSubcoreMesh(core_axis_name="c", subcore_axis_name="s")
    out_type = [jax.ShapeDtypeStruct((NTILES, NB_E, EB), jnp.float32)
                for _ in range(2)]
    scratch = [
        pltpu.VMEM((NB_E, EB), jnp.int32),
        pltpu.VMEM((NB_E, EB), jnp.int32),
        pltpu.VMEM((NB_E, EB), jnp.float32),
        pltpu.VMEM((NB_E, EB), jnp.float32),
        pltpu.VMEM((NB_E, EB), jnp.float32),
        pltpu.VMEM((EB, 16), jnp.float32),
        pltpu.VMEM((NROW, 16), jnp.float32),
        pltpu.VMEM((NPAD,), jnp.float32),
        pltpu.VMEM((NPAD,), jnp.float32),
        pltpu.VMEM_SHARED((NPAD, 16), jnp.float32),
        pltpu.VMEM_SHARED((NPAD, 16), jnp.float32),
        pltpu.VMEM_SHARED((NPAD,), jnp.float32),
        pltpu.VMEM_SHARED((NPAD,), jnp.float32),
    ]

    @functools.partial(
        pl.kernel, out_type=out_type, mesh=mesh, scratch_types=scratch,
        compiler_params=pltpu.CompilerParams(
            needs_layout_passes=False, use_tc_tiling_on_sc=False))
    def ka(src_h, dst_h, w_h, val1_h, val2_h,
           src_v, dst_v, val1_v, val2_v, w_v, zrow_v, dcol_v,
           dinvo_v, dinvi_v, deg2o, deg2i, dinv_o_sh, dinv_i_sh):
        c = lax.axis_index("c")
        s = lax.axis_index("s")
        iota16 = lax.iota(jnp.int32, 16)
        zc16 = jnp.zeros((16,), jnp.int32)
        z16 = jnp.zeros((16,), jnp.float32)
        pltpu.sync_copy(src_h.at[s], src_v)
        pltpu.sync_copy(dst_h.at[s], dst_v)

        if True:
            pltpu.sync_copy(w_h.at[s], w_v)

            def zrow_body(e, _):
                zrow_v[e, :] = z16
                return _
            lax.fori_loop(0, EB, zrow_body, None)
            row0 = s * NROW
            for t in range(NSUB):
                pltpu.sync_copy(zrow_v, deg2o.at[pl.ds(row0 + t * EB, EB)])
                pltpu.sync_copy(zrow_v, deg2i.at[pl.ds(row0 + t * EB, EB)])
            plsc.subcore_barrier()

            def degj(j, _):
                def dege(e16, _):
                    wv = w_v[j, pl.ds(e16 * 16, 16)]
                    ridx = e16 * 16 + iota16
                    plsc.store_scatter(zrow_v, [ridx, zc16], wv)
                    return _
                lax.fori_loop(0, EB // 16, dege, _)
                pltpu.sync_copy(zrow_v, deg2o.at[src_v.at[j]], add=True)
                pltpu.sync_copy(zrow_v, deg2i.at[dst_v.at[j]], add=True)
                return _
            lax.fori_loop(0, NB_E, degj, None)
            plsc.subcore_barrier()

            def invert(deg2, dinv_sh, dinv_v):
                pltpu.sync_copy(deg2.at[pl.ds(row0, NROW)], dcol_v)

                def cj(jv, _):
                    d = plsc.load_gather(dcol_v, [jv * 16 + iota16, zc16])
                    dv = jnp.where(d > 0.0, 1.0 / d, 0.0)
                    dinv_v[pl.ds(row0 + jv * 16, 16)] = dv
                    return _
                lax.fori_loop(0, NROW // 16, cj, None)
                pltpu.sync_copy(dinv_v.at[pl.ds(row0, NROW)],
                                dinv_sh.at[pl.ds(row0, NROW)])

            invert(deg2o, dinv_o_sh, dinvo_v)
            invert(deg2i, dinv_i_sh, dinvi_v)
            plsc.subcore_barrier()
            pltpu.sync_copy(dinv_o_sh, dinvo_v)
            pltpu.sync_copy(dinv_i_sh, dinvi_v)

            def vj(j, _):
                def ve(ev, _):
                    sl = pl.ds(ev * 16, 16)
                    si = src_v[j, sl]
                    di = dst_v[j, sl]
                    wv = w_v[j, sl]
                    val1_v[j, sl] = wv * plsc.load_gather(dinvo_v, [si])
                    val2_v[j, sl] = wv * plsc.load_gather(dinvi_v, [di])
                    return _
                return lax.fori_loop(0, EB // 16, ve, _)
            lax.fori_loop(0, NB_E, vj, None)

            @pl.when(c == 0)
            def _():
                pltpu.sync_copy(val1_v, val1_h.at[s])
                pltpu.sync_copy(val2_v, val2_h.at[s])

    return ka(src_t, dst_t, w_t)


def _sc_spmm(x0, src_t, dst_t, val1_t, val2_t):
    mesh = plsc.VectorSubcoreMesh(core_axis_name="c", subcore_axis_name="s")
    out_type = [jax.ShapeDtypeStruct((NQ, NPAD, FQ), jnp.float32)
                for _ in range(4)]
    scratch = [
        pltpu.VMEM((NB_E, EB), jnp.int32),
        pltpu.VMEM((NB_E, EB), jnp.int32),
        pltpu.VMEM((NB_E, EB), jnp.float32),
        pltpu.VMEM((NB_E, EB), jnp.float32),
        pltpu.VMEM((EB, FQ), jnp.float32),
        pltpu.VMEM((EB, FQ), jnp.float32),
        pltpu.VMEM_SHARED((NPAD, FQ), jnp.float32),
        pltpu.SemaphoreType.DMA,
        pltpu.SemaphoreType.DMA,
        pltpu.SemaphoreType.DMA,
        pltpu.SemaphoreType.DMA,
    ]

    @functools.partial(
        pl.kernel, out_type=out_type, mesh=mesh, scratch_types=scratch,
        compiler_params=pltpu.CompilerParams(
            needs_layout_passes=False, use_tc_tiling_on_sc=False))
    def kb(x0_h, src_h, dst_h, val1_h, val2_h, m1_h, m2_h, m3_h, m4_h,
           src_v, dst_v, val1_v, val2_v, buf_a, buf_b, acc,
           sem_a, sem_b, sem_sa, sem_sb):
        c = lax.axis_index("c")
        s = lax.axis_index("s")
        iota16 = lax.iota(jnp.int32, 16)
        z16 = jnp.zeros((16,), jnp.float32)
        pltpu.sync_copy(src_h.at[s], src_v)
        pltpu.sync_copy(dst_h.at[s], dst_v)
        pltpu.sync_copy(val1_h.at[s], val1_v)
        pltpu.sync_copy(val2_h.at[s], val2_v)

        if True:
            def zero_rows(e, _):
                for rsub in range(NVR):
                    buf_a[e, pl.ds(rsub * 16, 16)] = z16
                return _

            def scale_scatter(buf, val_v, double, gi):
                def sbody(e16, _):
                    v = val_v[gi, pl.ds(e16 * 16, 16)]
                    if double:
                        v = v + v
                    ridx = e16 * 16 + iota16

                    @plsc.parallel_loop(0, FQ, unroll=8)
                    def _(ci):
                        cidx = lax.broadcast(ci, (16,))
                        g = plsc.load_gather(buf, [ridx, cidx])
                        plsc.store_scatter(buf, [ridx, cidx], g * v)
                    return _
                lax.fori_loop(0, EB // 16, sbody, None)

            def do_pass(in_hq, gidx_v, sidx_v, val_v, double, out_hq, sub_hq):
                lax.fori_loop(0, EB, zero_rows, None)
                row0 = s * NROW
                for t in range(NSUB):
                    pltpu.sync_copy(buf_a, acc.at[pl.ds(row0 + t * EB, EB)])
                plsc.subcore_barrier()

                def gather(g, buf, sem):
                    return pltpu.make_async_copy(
                        in_hq.at[gidx_v.at[g]], buf, sem)

                def scatter(g, buf, sem):
                    return pltpu.make_async_copy(
                        buf, acc.at[sidx_v.at[g]], sem)

                gather(0, buf_a, sem_a).start()
                gather(1, buf_b, sem_b).start()

                def pair(g2, _):
                    g = g2 * 2
                    gather(g, buf_a, sem_a).wait()
                    scale_scatter(buf_a, val_v, double, g)
                    scatter(g, buf_a, sem_sa).start(add=True)
                    gather(g + 1, buf_b, sem_b).wait()
                    scale_scatter(buf_b, val_v, double, g + 1)
                    scatter(g, buf_a, sem_sa).wait()

                    @pl.when(g + 2 < NB_E)
                    def _():
                        gather(g + 2, buf_a, sem_a).start()
                    scatter(g + 1, buf_b, sem_sb).start(add=True)
                    scatter(g + 1, buf_b, sem_sb).wait()

                    @pl.when(g + 3 < NB_E)
                    def _():
                        gather(g + 3, buf_b, sem_b).start()
                    return _
                lax.fori_loop(0, NB_E // 2, pair, None)
                plsc.subcore_barrier()

                for t in range(NSUB):
                    r0 = row0 + t * EB
                    pltpu.sync_copy(acc.at[pl.ds(r0, EB)], buf_a)
                    if sub_hq is not None:
                        pltpu.sync_copy(sub_hq.at[pl.ds(r0, EB)], buf_b)

                        def sub_body(e, _):
                            for rsub in range(NVR):
                                sl = pl.ds(rsub * 16, 16)
                                buf_a[e, sl] = buf_a[e, sl] - buf_b[e, sl]
                            return _
                        lax.fori_loop(0, EB, sub_body, None)
                    pltpu.sync_copy(buf_a, out_hq.at[pl.ds(r0, EB)])
                plsc.subcore_barrier()

            def chunk(j, _):
                q = c * NQC + j
                x0q = x0_h.at[q]
                m1q = m1_h.at[q]
                m2q = m2_h.at[q]
                m3q = m3_h.at[q]
                m4q = m4_h.at[q]
                do_pass(x0q, src_v, dst_v, val1_v, False, m1q, None)
                do_pass(m1q, src_v, dst_v, val1_v, True, m2q, x0q)
                do_pass(m1q, dst_v, src_v, val2_v, False, m3q, None)
                do_pass(m3q, dst_v, src_v, val2_v, True, m4q, m1q)
                return _
            lax.fori_loop(0, NQC, chunk, None)

    return kb(x0, src_t, dst_t, val1_t, val2_t)


def _out_body(x0_ref, m1_ref, m2_ref, m3_ref, m4_ref, u_ref, hx_ref,
              wcat_ref, bg_ref, out_ref):
    wcat = wcat_ref[...]
    bg = bg_ref[...]
    for b in range(B):
        q, bq = divmod(b, BQ)
        c0 = bq * ISP
        mcat = jnp.concatenate(
            [x0_ref[q][:, c0:c0 + IS], m1_ref[q][:, c0:c0 + IS],
             m2_ref[q][:, c0:c0 + IS], m3_ref[q][:, c0:c0 + IS],
             m4_ref[q][:, c0:c0 + IS]], axis=1)
        cc = jnp.tanh(
            jnp.dot(mcat, wcat, preferred_element_type=jnp.float32) + bg)
        u_b = u_ref[b]
        hx_b = hx_ref[b]
        out_ref[b] = u_b * hx_b + (1.0 - u_b) * cc


def _output(x0, m1, m2, m3, m4, u3, hx_r, w_cat, b_g2):
    mspec = pl.BlockSpec((NQ, NBLK, FQ), lambda j: (0, j, 0))
    bspec = pl.BlockSpec((B, NBLK, UNITS), lambda j: (0, j, 0))
    return pl.pallas_call(
        _out_body,
        grid=(NBLOCKS,),
        in_specs=[mspec, mspec, mspec, mspec, mspec, bspec, bspec,
                  pl.BlockSpec((NUM_MATRICES * IS, UNITS), lambda j: (0, 0)),
                  pl.BlockSpec((1, UNITS), lambda j: (0, 0))],
        out_specs=bspec,
        out_shape=jax.ShapeDtypeStruct((B, N, UNITS), jnp.float32),
    )(x0, m1, m2, m3, m4, u3, hx_r, w_cat, b_g2)


def kernel(inputs, hx, edge_w, W_fc, b_fc, W_g, b_g, edge_src, edge_dst):
    inputs_r = inputs.reshape(B, N, IN_DIM)
    hx_r = hx.reshape(B, N, UNITS)
    b_fc2 = b_fc.reshape(1, 2 * UNITS)
    b_g2 = b_g.reshape(1, UNITS)
    w_cat = W_g.reshape(IS, NUM_MATRICES, UNITS).transpose(1, 0, 2)
    w_cat = w_cat.reshape(NUM_MATRICES * IS, UNITS)

    npad_e = EPAD - E
    pad_idx = (jnp.arange(npad_e, dtype=jnp.int32) * 41) % N
    src_t = jnp.concatenate([edge_src.astype(jnp.int32), pad_idx])
    src_t = src_t.reshape(NTILES, NB_E, EB)
    dst_t = jnp.concatenate([edge_dst.astype(jnp.int32), pad_idx])
    dst_t = dst_t.reshape(NTILES, NB_E, EB)
    w_t = jnp.concatenate(
        [edge_w, jnp.zeros((npad_e,), jnp.float32)]).reshape(NTILES, NB_E, EB)

    x0, u3 = _gates(inputs_r, hx_r, W_fc, b_fc2)
    val1_t, val2_t = _sc_degvals(src_t, dst_t, w_t)
    m1, m2, m3, m4 = _sc_spmm(x0, src_t, dst_t, val1_t, val2_t)
    out3 = _output(x0, m1, m2, m3, m4, u3, hx_r, w_cat, b_g2)
    return out3.reshape(B, N * UNITS)
